# baseline (speedup 1.0000x reference)
import functools

import jax
import jax.numpy as jnp
from jax import lax
from jax.experimental import pallas as pl
from jax.experimental.pallas import tpu as pltpu
from jax.experimental.pallas import tpu_sc as plsc

N_EMBED = 1024
EMBED_DIM = 64
BETA = 0.25
B = 16
HW = 1024
N_TOK = B * HW

NC, NS, L = 2, 16, 16
NW = NC * NS
TPW = N_TOK // NW
G = TPW // L


def _tc_main(z_ref, cb_ref, idx_ref, cbt_ref, loss_ref, loss_acc, cbsq_ref):
    b = pl.program_id(0)

    x = z_ref[0]
    cb = cb_ref[...]

    @pl.when(b == 0)
    def _precompute():
        cbsq_ref[...] = jnp.sum(cb * cb, axis=1)[None, :]
        cbt_ref[...] = jnp.transpose(cb, (1, 0))

    zf = jnp.transpose(x, (1, 0))
    a = jnp.sum(zf * zf, axis=1, keepdims=True)
    m = jnp.dot(zf, cb.T, preferred_element_type=jnp.float32)
    d = (a + cbsq_ref[...]) - 2.0 * m

    lane = jax.lax.broadcasted_iota(jnp.int32, (HW, N_EMBED), 1)
    dmin = jnp.min(d, axis=1, keepdims=True)
    idx = jnp.min(jnp.where(d == dmin, lane, N_EMBED), axis=1).astype(jnp.int32)
    idx_ref[0, 0] = idx

    sse = jnp.sum(dmin)

    @pl.when(b == 0)
    def _init():
        loss_acc[0] = sse

    @pl.when(b > 0)
    def _accum():
        loss_acc[0] += sse

    @pl.when(b == B - 1)
    def _finish():
        loss = (1.0 + BETA) * loss_acc[0] / jnp.float32(N_TOK * EMBED_DIM)
        lane_s = jax.lax.broadcasted_iota(jnp.int32, (1, 128), 1)
        loss_ref[...] = jnp.where(lane_s == 0, loss, 0.0)


def _sc_body(cbt_hbm, idx_hbm, out_hbm, counts_hbm,
             idx_v, cbt_v, blk_v, ones_v, zeros_v, shared):
    c = lax.axis_index("c")
    s = lax.axis_index("s")
    w = s * NC + c
    base = w * TPW

    pltpu.sync_copy(idx_hbm.at[pl.ds(base, TPW)], idx_v)
    pltpu.sync_copy(cbt_hbm, cbt_v)

    def _z(g, carry):
        zeros_v[pl.ds(g * L, L)] = jnp.zeros((L,), jnp.float32)
        return carry
    lax.fori_loop(0, N_EMBED // L, _z, 0)

    @pl.when(s == 0)
    def _init_hist():
        pltpu.sync_copy(zeros_v, shared)

    def _grp(g, carry):
        idx16 = idx_v[pl.ds(g * L, L)]
        ones_v[pl.ds(g * L, L)] = jnp.ones((L,), jnp.float32)
        for ch in range(EMBED_DIM):
            vals = plsc.load_gather(cbt_v, [idx16 + (ch * HW)])
            blk_v[ch, pl.ds(g * L, L)] = vals
        return carry
    lax.fori_loop(0, G, _grp, 0)

    bb = w // 2
    hh = w % 2
    pltpu.sync_copy(blk_v, out_hbm.at[bb, :, pl.ds(hh * TPW, TPW)])

    plsc.subcore_barrier()
    pltpu.sync_copy(ones_v, shared.at[idx_v], add=True)
    plsc.subcore_barrier()

    @pl.when(s == 0)
    def _dump():
        pltpu.sync_copy(shared, counts_hbm.at[c])


def _sc_stage(cbt, idx_flat):
    sck = pl.kernel(
        _sc_body,
        out_type=[
            jax.ShapeDtypeStruct((B, EMBED_DIM, HW), jnp.float32),
            jax.ShapeDtypeStruct((NC, N_EMBED), jnp.float32),
        ],
        mesh=plsc.VectorSubcoreMesh(core_axis_name="c", subcore_axis_name="s"),
        compiler_params=pltpu.CompilerParams(needs_layout_passes=False),
        scratch_types=[
            pltpu.VMEM((TPW,), jnp.int32),
            pltpu.VMEM((EMBED_DIM * HW,), jnp.float32),
            pltpu.VMEM((EMBED_DIM, TPW), jnp.float32),
            pltpu.VMEM((TPW,), jnp.float32),
            pltpu.VMEM((N_EMBED,), jnp.float32),
            pltpu.VMEM_SHARED((N_EMBED,), jnp.float32),
        ],
    )
    return sck(cbt, idx_flat)


def _tc_epi(counts_ref, out_ref):
    counts = counts_ref[0, :] + counts_ref[1, :]
    p = counts / jnp.float32(N_TOK)
    ent = jnp.sum(p * jnp.log(p + 1e-10))
    perp = jnp.exp(-ent)
    lane_s = jax.lax.broadcasted_iota(jnp.int32, (1, 128), 1)
    out_ref[...] = jnp.where(lane_s == 0, perp, 0.0)


def kernel(z, codebook):
    zr = z.reshape(B, EMBED_DIM, HW)

    idx3, cbt, loss_vec = pl.pallas_call(
        _tc_main,
        grid=(B,),
        in_specs=[
            pl.BlockSpec((1, EMBED_DIM, HW), lambda b: (b, 0, 0)),
            pl.BlockSpec((N_EMBED, EMBED_DIM), lambda b: (0, 0)),
        ],
        out_specs=[
            pl.BlockSpec((1, 1, HW), lambda b: (b, 0, 0)),
            pl.BlockSpec((EMBED_DIM, HW), lambda b: (0, 0)),
            pl.BlockSpec((1, 128), lambda b: (0, 0)),
        ],
        out_shape=[
            jax.ShapeDtypeStruct((B, 1, HW), jnp.int32),
            jax.ShapeDtypeStruct((EMBED_DIM, HW), jnp.float32),
            jax.ShapeDtypeStruct((1, 128), jnp.float32),
        ],
        scratch_shapes=[
            pltpu.SMEM((1,), jnp.float32),
            pltpu.VMEM((1, N_EMBED), jnp.float32),
        ],
    )(zr, codebook)

    encoding_indices = idx3.reshape(N_TOK)
    out, counts = _sc_stage(cbt.reshape(EMBED_DIM * HW), encoding_indices)
    perp_vec = pl.pallas_call(
        _tc_epi,
        in_specs=[pl.BlockSpec((NC, N_EMBED), lambda: (0, 0))],
        out_specs=pl.BlockSpec((1, 128), lambda: (0, 0)),
        out_shape=jax.ShapeDtypeStruct((1, 128), jnp.float32),
    )(counts)

    out4 = out.reshape(B, EMBED_DIM, 32, 32)
    loss = loss_vec[0, 0].reshape(())
    perplexity = perp_vec[0, 0].reshape(())
    return (out4, loss, perplexity, encoding_indices)


# speedup vs baseline: 1.5551x; 1.5551x over previous
import jax
import jax.numpy as jnp
from jax.experimental import pallas as pl
from jax.experimental.pallas import tpu as pltpu

N_EMBED = 1024
EMBED_DIM = 64
BETA = 0.25
B = 16
HW = 1024
N_TOK = B * HW


def _vq_kernel(z_ref, cb_ref, out_ref, idx_ref, scalars_ref,
               counts_acc, loss_acc, cbsq_ref):
    b = pl.program_id(0)

    x = z_ref[0]
    cb = cb_ref[...]

    @pl.when(b == 0)
    def _precompute():
        cbsq_ref[...] = jnp.sum(cb * cb, axis=1)[None, :]

    zf = jnp.transpose(x, (1, 0))

    a = jnp.sum(zf * zf, axis=1, keepdims=True)
    cb_sq = cbsq_ref[...]
    m = jnp.dot(zf, cb.T, preferred_element_type=jnp.float32)
    d = (a + cb_sq) - 2.0 * m

    lane = jax.lax.broadcasted_iota(jnp.int32, (HW, N_EMBED), 1)
    dmin = jnp.min(d, axis=1, keepdims=True)
    at_min = d == dmin
    idx = jnp.min(jnp.where(at_min, lane, N_EMBED), axis=1).astype(jnp.int32)
    idx_ref[0, 0] = idx

    e = (lane == idx[:, None]).astype(jnp.bfloat16)
    cb_bf = cb.astype(jnp.bfloat16)

    zq_t = jax.lax.dot_general(
        cb_bf, e, (((0,), (1,)), ((), ())),
        preferred_element_type=jnp.float32)
    out_ref[0] = zq_t

    sse = jnp.sum(dmin)
    ones_row = jnp.ones((8, HW), jnp.bfloat16)
    counts = jnp.dot(ones_row, e, preferred_element_type=jnp.float32)[:1]

    @pl.when(b == 0)
    def _init():
        loss_acc[0] = sse
        counts_acc[...] = counts

    @pl.when(b > 0)
    def _accum():
        loss_acc[0] += sse
        counts_acc[...] += counts

    @pl.when(b == B - 1)
    def _finish():
        loss = (1.0 + BETA) * loss_acc[0] / jnp.float32(N_TOK * EMBED_DIM)
        p = counts_acc[...] / jnp.float32(N_TOK)
        ent = jnp.sum(p * jnp.log(p + 1e-10))
        perp = jnp.exp(-ent)
        lane_s = jax.lax.broadcasted_iota(jnp.int32, (1, 128), 1)
        vec = jnp.where(lane_s == 0, loss,
                        jnp.where(lane_s == 1, perp, 0.0))
        scalars_ref[...] = vec


def kernel(z, codebook):
    zr = z.reshape(B, EMBED_DIM, HW)

    out, idx, scalars = pl.pallas_call(
        _vq_kernel,
        grid=(B,),
        in_specs=[
            pl.BlockSpec((1, EMBED_DIM, HW), lambda b: (b, 0, 0)),
            pl.BlockSpec((N_EMBED, EMBED_DIM), lambda b: (0, 0)),
        ],
        out_specs=[
            pl.BlockSpec((1, EMBED_DIM, HW), lambda b: (b, 0, 0)),
            pl.BlockSpec((1, 1, HW), lambda b: (b, 0, 0)),
            pl.BlockSpec((1, 128), lambda b: (0, 0)),
        ],
        out_shape=[
            jax.ShapeDtypeStruct((B, EMBED_DIM, HW), jnp.float32),
            jax.ShapeDtypeStruct((B, 1, HW), jnp.int32),
            jax.ShapeDtypeStruct((1, 128), jnp.float32),
        ],
        scratch_shapes=[
            pltpu.VMEM((1, N_EMBED), jnp.float32),
            pltpu.SMEM((1,), jnp.float32),
            pltpu.VMEM((1, N_EMBED), jnp.float32),
        ],
    )(zr, codebook)

    out4 = out.reshape(B, EMBED_DIM, 32, 32)
    loss = scalars[0, 0].reshape(())
    perplexity = scalars[0, 1].reshape(())
    encoding_indices = idx.reshape(N_TOK)
    return (out4, loss, perplexity, encoding_indices)
